# Initial kernel scaffold; baseline (speedup 1.0000x reference)
import jax
import jax.numpy as jnp
from jax.experimental import pallas as pl


def kernel(x1, x2, W_att, V_att, Wt_att, U_att, b_att, V_ntn, W_ntn, b_ntn, proj0, proj1, proj2, proj3):
    raise NotImplementedError("write your pallas kernel here")


# speedup vs baseline: 1.1419x; 1.1419x over previous
import functools

import jax
import jax.numpy as jnp
from jax.experimental import pallas as pl
from jax.experimental.pallas import tpu as pltpu

_D = 128
_H = 2
_EPS = 1e-12
_NBLK = 5000

_DN_NT = (((1,), (1,)), ((), ()))
_DN_TN = (((0,), (0,)), ((), ()))
_DN_NN = (((1,), (0,)), ((), ()))


def _dot(a, b, dn, prec=None):
    return jax.lax.dot_general(a, b, dn, precision=prec,
                               preferred_element_type=jnp.float32)


def _sum_body(x1_ref, x2_ref, s_ref):
    j = pl.program_id(1)

    @pl.when(j == 0)
    def _():
        s_ref[...] = jnp.zeros_like(s_ref)

    s1 = jnp.sum(x1_ref[0], axis=0, keepdims=True)
    s2 = jnp.sum(x2_ref[0], axis=0, keepdims=True)
    s_ref[...] += jnp.concatenate([s1, s2], axis=0).reshape(s_ref.shape)


def _head_params(m, w_i, wt_i, vb_i, b_i):
    h = jnp.tanh(_dot(m, w_i, _DN_NN))
    w2 = _dot(h, wt_i, _DN_NT)
    h16 = h.astype(jnp.bfloat16).astype(jnp.float32)
    vb16 = vb_i.astype(jnp.bfloat16).astype(jnp.float32)
    c = jnp.sum(h16 * vb16, axis=1, keepdims=True) + b_i
    return h, w2, c


def _att1_both(xb, s_row, n_total, w, wt, va, vb, b_ref, u_ref):
    m = s_row * (1.0 / n_total)
    rows = []
    cs = []
    for i in range(_H):
        _, w2, c = _head_params(m, w[i], wt[i], vb[i:i + 1], b_ref[i])
        rows.append(va[i:i + 1])
        rows.append(w2)
        cs.append(c)
    rmat = jnp.concatenate(rows, axis=0)
    tt = _dot(xb, rmat, _DN_NT)
    atts = []
    for i in range(_H):
        t = tt[:, 2 * i:2 * i + 1] + tt[:, 2 * i + 1:2 * i + 2] + cs[i]
        t = t / jnp.maximum(jnp.abs(t), _EPS)
        atts.append(jax.nn.sigmoid(t) * u_ref[i])
    return jnp.concatenate(atts, axis=1)


def _pool1_body(x1_ref, x2_ref, s_ref, w_ref, wt_ref, va_ref, vb_ref,
                b_ref, u_ref, o1_ref, *, n_total):
    j = pl.program_id(1)
    w = w_ref[...]
    wt = wt_ref[...]
    va = va_ref[...]
    vb = vb_ref[...]
    s_all = s_ref[0]

    parts = []
    for side, x_ref in ((0, x1_ref), (1, x2_ref)):
        xb = x_ref[0]
        att1 = _att1_both(xb, s_all[side:side + 1], n_total,
                          w, wt, va, vb, b_ref, u_ref)
        parts.append(_dot(att1, xb, _DN_TN,
                          jax.lax.Precision.HIGHEST))

    blk = jnp.concatenate(parts, axis=0).reshape(o1_ref.shape)

    @pl.when(j == 0)
    def _():
        o1_ref[...] = jnp.zeros_like(o1_ref)

    o1_ref[...] += blk


def _pool2_body(x1_ref, x2_ref, s_ref, o1in_ref, w_ref, wt_ref, va_ref,
                vb_ref, b_ref, u_ref, o2_ref, *, n_total):
    j = pl.program_id(1)
    w = w_ref[...]
    wt = wt_ref[...]
    va = va_ref[...]
    vb = vb_ref[...]
    s_all = s_ref[0]
    o1_all = o1in_ref[0]

    parts = []
    for side, x_ref in ((0, x1_ref), (1, x2_ref)):
        xb = x_ref[0]
        att1 = _att1_both(xb, s_all[side:side + 1], n_total,
                          w, wt, va, vb, b_ref, u_ref)
        xc_cols = []
        att2_cols = []
        for i in range(_H):
            xc = att1[:, i:i + 1] * xb
            m2 = o1_all[side, i:i + 1] * (1.0 / n_total)
            _, w2, c2 = _head_params(m2, w[i], wt[i], vb[i:i + 1],
                                     b_ref[i])
            r2 = jnp.concatenate([va[i:i + 1], w2], axis=0)
            tt = _dot(xc, r2, _DN_NT)
            t2 = tt[:, 0:1] + tt[:, 1:2] + c2
            t2 = t2 / jnp.maximum(jnp.abs(t2), _EPS)
            att2_cols.append(jax.nn.sigmoid(t2) * u_ref[i])
            xc_cols.append(xc)
        att2 = jnp.concatenate(att2_cols, axis=1)
        xc_all = jnp.concatenate(xc_cols, axis=1)
        pooled = _dot(att2, xc_all, _DN_TN)
        parts.append(jnp.concatenate(
            [pooled[i:i + 1, i * _D:(i + 1) * _D] for i in range(_H)],
            axis=0))

    blk = jnp.concatenate(parts, axis=0).reshape(o2_ref.shape)

    @pl.when(j == 0)
    def _():
        o2_ref[...] = jnp.zeros_like(o2_ref)

    o2_ref[...] += blk


def _head_body(g1_ref, g2_ref, vn_ref, wn_ref, bn_ref, p0_ref, p1_ref,
               p2_ref, p3_ref, out_ref):
    g1 = g1_ref[...]
    g2 = g2_ref[...]
    din2 = g1.shape[1]
    vn = vn_ref[...]
    van = vn[:, :din2]
    vbn = vn[:, din2:]
    fmap = vn.shape[0]

    cols = []
    for f in range(fmap):
        uf = _dot(g1, wn_ref[f], _DN_NN)
        cols.append(jnp.sum(uf * g2, axis=1, keepdims=True))
    s_bil = jnp.concatenate(cols, axis=1)

    s = _dot(g1, van, _DN_NT) + _dot(g2, vbn, _DN_NT) + s_bil + bn_ref[...]
    s = s / jnp.maximum(jnp.sum(jnp.abs(s), axis=1, keepdims=True), _EPS)
    y = jax.nn.relu(s)
    for p_ref in (p0_ref, p1_ref, p2_ref, p3_ref):
        y = _dot(y, p_ref[...], _DN_NT)
    out_ref[...] = y


def kernel(x1, x2, W_att, V_att, Wt_att, U_att, b_att,
           V_ntn, W_ntn, b_ntn, proj0, proj1, proj2, proj3):
    B, N, D = x1.shape
    nb = _NBLK if N % _NBLK == 0 else N
    num_blocks = N // nb

    Va = V_att[:, 0, :D]
    Vb = V_att[:, 0, D:]
    Wt = Wt_att[:, 0]
    u_vec = U_att[:, 0, 0]
    b_vec = b_att[:, 0]

    grid = (B, num_blocks)
    x_spec = pl.BlockSpec((1, nb, D), lambda b, j: (b, j, 0))
    w_specs = [
        pl.BlockSpec(W_att.shape, lambda b, j: (0, 0, 0)),
        pl.BlockSpec(Wt.shape, lambda b, j: (0, 0, 0)),
        pl.BlockSpec(Va.shape, lambda b, j: (0, 0)),
        pl.BlockSpec(Vb.shape, lambda b, j: (0, 0)),
        pl.BlockSpec(memory_space=pltpu.SMEM),
        pl.BlockSpec(memory_space=pltpu.SMEM),
    ]
    cparams = pltpu.CompilerParams(
        dimension_semantics=("parallel", "arbitrary"))

    sums = pl.pallas_call(
        _sum_body,
        out_shape=jax.ShapeDtypeStruct((B, 2, D), jnp.float32),
        grid=grid,
        in_specs=[x_spec, x_spec],
        out_specs=pl.BlockSpec((1, 2, D), lambda b, j: (b, 0, 0)),
        compiler_params=cparams,
        name="anpm_sums",
    )(x1, x2)

    out1 = pl.pallas_call(
        functools.partial(_pool1_body, n_total=N),
        out_shape=jax.ShapeDtypeStruct((B, 2, _H, D), jnp.float32),
        grid=grid,
        in_specs=[x_spec, x_spec,
                  pl.BlockSpec((1, 2, D), lambda b, j: (b, 0, 0))] + w_specs,
        out_specs=pl.BlockSpec((1, 2, _H, D), lambda b, j: (b, 0, 0, 0)),
        compiler_params=cparams,
        name="anpm_pool1",
    )(x1, x2, sums, W_att, Wt, Va, Vb, b_vec, u_vec)

    out2 = pl.pallas_call(
        functools.partial(_pool2_body, n_total=N),
        out_shape=jax.ShapeDtypeStruct((B, 2, _H, D), jnp.float32),
        grid=grid,
        in_specs=[x_spec, x_spec,
                  pl.BlockSpec((1, 2, D), lambda b, j: (b, 0, 0)),
                  pl.BlockSpec((1, 2, _H, D), lambda b, j: (b, 0, 0, 0))]
                 + w_specs,
        out_specs=pl.BlockSpec((1, 2, _H, D), lambda b, j: (b, 0, 0, 0)),
        compiler_params=cparams,
        name="anpm_pool2",
    )(x1, x2, sums, out1, W_att, Wt, Va, Vb, b_vec, u_vec)

    g = out2.reshape(B, 2, _H * D)
    g1 = g[:, 0]
    g2 = g[:, 1]

    out = pl.pallas_call(
        _head_body,
        out_shape=jax.ShapeDtypeStruct((B, 1), jnp.float32),
        name="anpm_head",
    )(g1, g2, V_ntn, W_ntn, b_ntn.reshape(1, -1), proj0, proj1, proj2, proj3)
    return out


# speedup vs baseline: 1.2600x; 1.1034x over previous
import functools

import jax
import jax.numpy as jnp
from jax.experimental import pallas as pl
from jax.experimental.pallas import tpu as pltpu

_D = 128
_H = 2
_EPS = 1e-12
_NBLK = 10000

_DN_NT = (((1,), (1,)), ((), ()))
_DN_TN = (((0,), (0,)), ((), ()))
_DN_NN = (((1,), (0,)), ((), ()))


def _dot(a, b, dn, prec=None):
    return jax.lax.dot_general(a, b, dn, precision=prec,
                               preferred_element_type=jnp.float32)


def _sum_body(x1_ref, x2_ref, s_ref):
    j = pl.program_id(1)

    @pl.when(j == 0)
    def _():
        s_ref[...] = jnp.zeros_like(s_ref)

    s1 = jnp.sum(x1_ref[0], axis=0, keepdims=True)
    s2 = jnp.sum(x2_ref[0], axis=0, keepdims=True)
    s_ref[...] += jnp.concatenate([s1, s2], axis=0).reshape(s_ref.shape)


def _head_params(m, w_i, wt_i, vb_i, b_i):
    h = jnp.tanh(_dot(m, w_i, _DN_NN))
    w2 = _dot(h, wt_i, _DN_NT)
    h16 = h.astype(jnp.bfloat16).astype(jnp.float32)
    vb16 = vb_i.astype(jnp.bfloat16).astype(jnp.float32)
    c = jnp.sum(h16 * vb16, axis=1, keepdims=True) + b_i
    return h, w2, c


def _att1_both(xb, s_row, n_total, w, wt, va, vb, b_ref, u_ref):
    m = s_row * (1.0 / n_total)
    rows = []
    cs = []
    for i in range(_H):
        _, w2, c = _head_params(m, w[i], wt[i], vb[i:i + 1], b_ref[i])
        rows.append(va[i:i + 1])
        rows.append(w2)
        cs.append(c)
    rmat = jnp.concatenate(rows, axis=0)
    tt = _dot(xb, rmat, _DN_NT)
    atts = []
    for i in range(_H):
        t = tt[:, 2 * i:2 * i + 1] + tt[:, 2 * i + 1:2 * i + 2] + cs[i]
        t = t / jnp.maximum(jnp.abs(t), _EPS)
        atts.append(jax.nn.sigmoid(t) * u_ref[i])
    return jnp.concatenate(atts, axis=1)


def _pool1_body(x1_ref, x2_ref, s_ref, w_ref, wt_ref, va_ref, vb_ref,
                b_ref, u_ref, o1_ref, *, n_total):
    j = pl.program_id(1)
    w = w_ref[...]
    wt = wt_ref[...]
    va = va_ref[...]
    vb = vb_ref[...]
    s_all = s_ref[0]

    parts = []
    for side, x_ref in ((0, x1_ref), (1, x2_ref)):
        xb = x_ref[0]
        att1 = _att1_both(xb, s_all[side:side + 1], n_total,
                          w, wt, va, vb, b_ref, u_ref)
        parts.append(_dot(att1, xb, _DN_TN,
                          jax.lax.Precision.HIGHEST))

    blk = jnp.concatenate(parts, axis=0).reshape(o1_ref.shape)

    @pl.when(j == 0)
    def _():
        o1_ref[...] = jnp.zeros_like(o1_ref)

    o1_ref[...] += blk


def _pool2_body(x1_ref, x2_ref, s_ref, o1in_ref, w_ref, wt_ref, va_ref,
                vb_ref, b_ref, u_ref, o2_ref, *, n_total):
    j = pl.program_id(1)
    w = w_ref[...]
    wt = wt_ref[...]
    va = va_ref[...]
    vb = vb_ref[...]
    s_all = s_ref[0]
    o1_all = o1in_ref[0]

    parts = []
    for side, x_ref in ((0, x1_ref), (1, x2_ref)):
        xb = x_ref[0]
        att1 = _att1_both(xb, s_all[side:side + 1], n_total,
                          w, wt, va, vb, b_ref, u_ref)
        xc_cols = []
        att2_cols = []
        for i in range(_H):
            xc = att1[:, i:i + 1] * xb
            m2 = o1_all[side, i:i + 1] * (1.0 / n_total)
            _, w2, c2 = _head_params(m2, w[i], wt[i], vb[i:i + 1],
                                     b_ref[i])
            r2 = jnp.concatenate([va[i:i + 1], w2], axis=0)
            tt = _dot(xc, r2, _DN_NT)
            t2 = tt[:, 0:1] + tt[:, 1:2] + c2
            t2 = t2 / jnp.maximum(jnp.abs(t2), _EPS)
            att2_cols.append(jax.nn.sigmoid(t2) * u_ref[i])
            xc_cols.append(xc)
        att2 = jnp.concatenate(att2_cols, axis=1)
        xc_all = jnp.concatenate(xc_cols, axis=1)
        pooled = _dot(att2, xc_all, _DN_TN)
        parts.append(jnp.concatenate(
            [pooled[i:i + 1, i * _D:(i + 1) * _D] for i in range(_H)],
            axis=0))

    blk = jnp.concatenate(parts, axis=0).reshape(o2_ref.shape)

    @pl.when(j == 0)
    def _():
        o2_ref[...] = jnp.zeros_like(o2_ref)

    o2_ref[...] += blk


def _head_body(g1_ref, g2_ref, vn_ref, wn_ref, bn_ref, p0_ref, p1_ref,
               p2_ref, p3_ref, out_ref):
    g1 = g1_ref[...]
    g2 = g2_ref[...]
    din2 = g1.shape[1]
    vn = vn_ref[...]
    van = vn[:, :din2]
    vbn = vn[:, din2:]
    fmap = vn.shape[0]

    cols = []
    for f in range(fmap):
        uf = _dot(g1, wn_ref[f], _DN_NN)
        cols.append(jnp.sum(uf * g2, axis=1, keepdims=True))
    s_bil = jnp.concatenate(cols, axis=1)

    s = _dot(g1, van, _DN_NT) + _dot(g2, vbn, _DN_NT) + s_bil + bn_ref[...]
    s = s / jnp.maximum(jnp.sum(jnp.abs(s), axis=1, keepdims=True), _EPS)
    y = jax.nn.relu(s)
    for p_ref in (p0_ref, p1_ref, p2_ref, p3_ref):
        y = _dot(y, p_ref[...], _DN_NT)
    out_ref[...] = y


def kernel(x1, x2, W_att, V_att, Wt_att, U_att, b_att,
           V_ntn, W_ntn, b_ntn, proj0, proj1, proj2, proj3):
    B, N, D = x1.shape
    nb = _NBLK if N % _NBLK == 0 else N
    num_blocks = N // nb

    Va = V_att[:, 0, :D]
    Vb = V_att[:, 0, D:]
    Wt = Wt_att[:, 0]
    u_vec = U_att[:, 0, 0]
    b_vec = b_att[:, 0]

    grid = (B, num_blocks)
    x_spec = pl.BlockSpec((1, nb, D), lambda b, j: (b, j, 0))
    w_specs = [
        pl.BlockSpec(W_att.shape, lambda b, j: (0, 0, 0)),
        pl.BlockSpec(Wt.shape, lambda b, j: (0, 0, 0)),
        pl.BlockSpec(Va.shape, lambda b, j: (0, 0)),
        pl.BlockSpec(Vb.shape, lambda b, j: (0, 0)),
        pl.BlockSpec(memory_space=pltpu.SMEM),
        pl.BlockSpec(memory_space=pltpu.SMEM),
    ]
    cparams = pltpu.CompilerParams(
        dimension_semantics=("parallel", "arbitrary"))

    sums = pl.pallas_call(
        _sum_body,
        out_shape=jax.ShapeDtypeStruct((B, 2, D), jnp.float32),
        grid=grid,
        in_specs=[x_spec, x_spec],
        out_specs=pl.BlockSpec((1, 2, D), lambda b, j: (b, 0, 0)),
        compiler_params=cparams,
        name="anpm_sums",
    )(x1, x2)

    out1 = pl.pallas_call(
        functools.partial(_pool1_body, n_total=N),
        out_shape=jax.ShapeDtypeStruct((B, 2, _H, D), jnp.float32),
        grid=grid,
        in_specs=[x_spec, x_spec,
                  pl.BlockSpec((1, 2, D), lambda b, j: (b, 0, 0))] + w_specs,
        out_specs=pl.BlockSpec((1, 2, _H, D), lambda b, j: (b, 0, 0, 0)),
        compiler_params=cparams,
        name="anpm_pool1",
    )(x1, x2, sums, W_att, Wt, Va, Vb, b_vec, u_vec)

    out2 = pl.pallas_call(
        functools.partial(_pool2_body, n_total=N),
        out_shape=jax.ShapeDtypeStruct((B, 2, _H, D), jnp.float32),
        grid=grid,
        in_specs=[x_spec, x_spec,
                  pl.BlockSpec((1, 2, D), lambda b, j: (b, 0, 0)),
                  pl.BlockSpec((1, 2, _H, D), lambda b, j: (b, 0, 0, 0))]
                 + w_specs,
        out_specs=pl.BlockSpec((1, 2, _H, D), lambda b, j: (b, 0, 0, 0)),
        compiler_params=cparams,
        name="anpm_pool2",
    )(x1, x2, sums, out1, W_att, Wt, Va, Vb, b_vec, u_vec)

    g = out2.reshape(B, 2, _H * D)
    g1 = g[:, 0]
    g2 = g[:, 1]

    out = pl.pallas_call(
        _head_body,
        out_shape=jax.ShapeDtypeStruct((B, 1), jnp.float32),
        name="anpm_head",
    )(g1, g2, V_ntn, W_ntn, b_ntn.reshape(1, -1), proj0, proj1, proj2, proj3)
    return out
